# baseline (device time: 419698 ns/iter reference)
import jax
import jax.numpy as jnp
from jax import lax
from jax.experimental import pallas as pl
from jax.experimental.pallas import tpu as pltpu

N_DEV = 16


def kernel(x, w_mat):
    m_per, k = x.shape
    _, n_per = w_mat.shape
    m_tot = N_DEV * m_per

    xb = x.astype(jnp.bfloat16)
    wb = w_mat.astype(jnp.bfloat16)

    def body(x_ref, w_ref, out_ref, comm_ref, gbuf_ref, abuf_ref,
             send_sems, recv_sems, a_send_sems, a_recv_sems):
        my = lax.axis_index("i")
        left = (my - 1) % N_DEV
        right = (my + 1) % N_DEV

        barrier_sem = pltpu.get_barrier_semaphore()
        for nbr in (left, right):
            pl.semaphore_signal(
                barrier_sem, inc=1,
                device_id=(nbr,), device_id_type=pl.DeviceIdType.MESH,
            )
        pl.semaphore_wait(barrier_sem, 2)

        def gemm(rel):
            origin = (my + rel) % N_DEV
            src = x_ref[...] if rel == 0 else comm_ref[rel]
            blk = jnp.dot(src, w_ref[...], preferred_element_type=jnp.float32)
            out_ref[pl.ds(origin * m_per, m_per), :] = blk

        gemm(0)

        for h in range(N_DEV - 1):
            recv_slot = 15 - h
            src = x_ref if h == 0 else comm_ref.at[16 - h]
            rdma = pltpu.make_async_remote_copy(
                src_ref=src,
                dst_ref=comm_ref.at[recv_slot],
                send_sem=send_sems.at[h],
                recv_sem=recv_sems.at[h],
                device_id=(right,),
                device_id_type=pl.DeviceIdType.MESH,
            )
            rdma.start()
            rdma.wait()
            gemm(recv_slot)

        local_amax = jnp.max(jnp.abs(out_ref[...]))
        abuf_ref[...] = jnp.full((8, 128), local_amax, jnp.float32)
        gbuf_ref[0] = jnp.full((8, 128), local_amax, jnp.float32)

        amax_rdmas = []
        for dist in range(1, N_DEV):
            r = pltpu.make_async_remote_copy(
                src_ref=abuf_ref,
                dst_ref=gbuf_ref.at[N_DEV - dist],
                send_sem=a_send_sems.at[dist - 1],
                recv_sem=a_recv_sems.at[dist - 1],
                device_id=((my + dist) % N_DEV,),
                device_id_type=pl.DeviceIdType.MESH,
            )
            r.start()
            amax_rdmas.append(r)
        for r in amax_rdmas:
            r.wait()

        gmax = jnp.max(gbuf_ref[...])
        scale = gmax / 448.0
        q = (out_ref[...] / scale).astype(jnp.float8_e4m3fn)
        out_ref[...] = q.astype(jnp.float32) * scale

    return pl.pallas_call(
        body,
        out_shape=jax.ShapeDtypeStruct((m_tot, n_per), jnp.float32),
        in_specs=[
            pl.BlockSpec(memory_space=pltpu.VMEM),
            pl.BlockSpec(memory_space=pltpu.VMEM),
        ],
        out_specs=pl.BlockSpec(memory_space=pltpu.VMEM),
        scratch_shapes=[
            pltpu.VMEM((N_DEV, m_per, k), jnp.bfloat16),
            pltpu.VMEM((N_DEV, 8, 128), jnp.float32),
            pltpu.VMEM((8, 128), jnp.float32),
            pltpu.SemaphoreType.DMA((N_DEV - 1,)),
            pltpu.SemaphoreType.DMA((N_DEV - 1,)),
            pltpu.SemaphoreType.DMA((N_DEV - 1,)),
            pltpu.SemaphoreType.DMA((N_DEV - 1,)),
        ],
        compiler_params=pltpu.CompilerParams(
            collective_id=0,
            vmem_limit_bytes=100 * 1024 * 1024,
        ),
    )(xb, wb)


# device time: 226701 ns/iter; 1.8513x vs baseline; 1.8513x over previous
import jax
import jax.numpy as jnp
from jax import lax
from jax.experimental import pallas as pl
from jax.experimental.pallas import tpu as pltpu

N_DEV = 16
R_HOPS = 8
L_HOPS = 7


def kernel(x, w_mat):
    m_per, k = x.shape
    _, n_per = w_mat.shape
    m_tot = N_DEV * m_per

    xb = x.astype(jnp.bfloat16)
    wb = w_mat.astype(jnp.bfloat16)

    def body(x_ref, w_ref, out_ref, comm_ref, gbuf_ref, abuf_ref,
             r_send_sems, r_recv_sems, l_send_sems, l_recv_sems,
             a_send_sems, a_recv_sems):
        my = lax.axis_index("i")
        left = (my - 1) % N_DEV
        right = (my + 1) % N_DEV

        barrier_sem = pltpu.get_barrier_semaphore()
        for nbr in (left, right):
            pl.semaphore_signal(
                barrier_sem, inc=1,
                device_id=(nbr,), device_id_type=pl.DeviceIdType.MESH,
            )
        pl.semaphore_wait(barrier_sem, 2)

        def gemm(rel):
            origin = (my + rel) % N_DEV
            src = x_ref[...] if rel == 0 else comm_ref[rel]
            blk = jnp.dot(src, w_ref[...], preferred_element_type=jnp.float32)
            out_ref[pl.ds(origin * m_per, m_per), :] = blk
            return jnp.max(jnp.abs(blk))

        def mrc(src, dst_slot, send_sem, recv_sem, dev):
            return pltpu.make_async_remote_copy(
                src_ref=src,
                dst_ref=comm_ref.at[dst_slot],
                send_sem=send_sem,
                recv_sem=recv_sem,
                device_id=(dev,),
                device_id_type=pl.DeviceIdType.MESH,
            )

        r_rdmas = [mrc(x_ref, 15, r_send_sems.at[0], r_recv_sems.at[0], right)]
        l_rdmas = [mrc(x_ref, 1, l_send_sems.at[0], l_recv_sems.at[0], left)]
        r_rdmas[0].start()
        l_rdmas[0].start()

        amax = gemm(0)

        for h in range(R_HOPS):
            r_rdmas[h].wait_recv()
            if h + 1 < R_HOPS:
                nxt = mrc(comm_ref.at[15 - h], 14 - h,
                          r_send_sems.at[h + 1], r_recv_sems.at[h + 1], right)
                nxt.start()
                r_rdmas.append(nxt)
            if h < L_HOPS:
                l_rdmas[h].wait_recv()
                if h + 1 < L_HOPS:
                    nxt = mrc(comm_ref.at[1 + h], 2 + h,
                              l_send_sems.at[h + 1], l_recv_sems.at[h + 1], left)
                    nxt.start()
                    l_rdmas.append(nxt)
            amax = jnp.maximum(amax, gemm(15 - h))
            if h < L_HOPS:
                amax = jnp.maximum(amax, gemm(1 + h))

        for r in r_rdmas + l_rdmas:
            r.wait_send()

        abuf_ref[...] = jnp.full((8, 128), amax, jnp.float32)
        gbuf_ref[0] = jnp.full((8, 128), amax, jnp.float32)

        amax_rdmas = []
        for dist in range(1, N_DEV):
            r = pltpu.make_async_remote_copy(
                src_ref=abuf_ref,
                dst_ref=gbuf_ref.at[N_DEV - dist],
                send_sem=a_send_sems.at[dist - 1],
                recv_sem=a_recv_sems.at[dist - 1],
                device_id=((my + dist) % N_DEV,),
                device_id_type=pl.DeviceIdType.MESH,
            )
            r.start()
            amax_rdmas.append(r)
        for r in amax_rdmas:
            r.wait()

        gmax = jnp.max(gbuf_ref[...])
        scale = gmax / 448.0
        q = (out_ref[...] / scale).astype(jnp.float8_e4m3fn)
        out_ref[...] = q.astype(jnp.float32) * scale

    return pl.pallas_call(
        body,
        out_shape=jax.ShapeDtypeStruct((m_tot, n_per), jnp.float32),
        in_specs=[
            pl.BlockSpec(memory_space=pltpu.VMEM),
            pl.BlockSpec(memory_space=pltpu.VMEM),
        ],
        out_specs=pl.BlockSpec(memory_space=pltpu.VMEM),
        scratch_shapes=[
            pltpu.VMEM((N_DEV, m_per, k), jnp.bfloat16),
            pltpu.VMEM((N_DEV, 8, 128), jnp.float32),
            pltpu.VMEM((8, 128), jnp.float32),
            pltpu.SemaphoreType.DMA((R_HOPS,)),
            pltpu.SemaphoreType.DMA((R_HOPS,)),
            pltpu.SemaphoreType.DMA((L_HOPS,)),
            pltpu.SemaphoreType.DMA((L_HOPS,)),
            pltpu.SemaphoreType.DMA((N_DEV - 1,)),
            pltpu.SemaphoreType.DMA((N_DEV - 1,)),
        ],
        compiler_params=pltpu.CompilerParams(
            collective_id=0,
            vmem_limit_bytes=100 * 1024 * 1024,
        ),
    )(xb, wb)


# device time: 209942 ns/iter; 1.9991x vs baseline; 1.0798x over previous
import jax
import jax.numpy as jnp
from jax import lax
from jax.experimental import pallas as pl
from jax.experimental.pallas import tpu as pltpu

N_DEV = 16
RA, LA = 8, 7
RB, LB = 7, 8


def kernel(x, w_mat):
    m_per, k = x.shape
    _, n_per = w_mat.shape
    m_tot = N_DEV * m_per
    mh = m_per // 2

    xb = x.astype(jnp.bfloat16)
    wb = w_mat.astype(jnp.bfloat16)

    def body(x_ref, w_ref, out_ref, commA_ref, commB_ref,
             gbuf_ref, abuf_ref, copy_sems,
             ra_ss, ra_rs, rb_ss, rb_rs, la_ss, la_rs, lb_ss, lb_rs,
             a_send_sems, a_recv_sems):
        my = lax.axis_index("i")
        left = (my - 1) % N_DEV
        right = (my + 1) % N_DEV

        barrier_sem = pltpu.get_barrier_semaphore()
        for nbr in (left, right):
            pl.semaphore_signal(
                barrier_sem, inc=1,
                device_id=(nbr,), device_id_type=pl.DeviceIdType.MESH,
            )
        pl.semaphore_wait(barrier_sem, 2)

        cpa = pltpu.make_async_copy(
            x_ref.at[pl.ds(0, mh)], commA_ref.at[0], copy_sems.at[0])
        cpb = pltpu.make_async_copy(
            x_ref.at[pl.ds(mh, mh)], commB_ref.at[0], copy_sems.at[1])
        cpa.start()
        cpb.start()
        cpa.wait()
        cpb.wait()

        def gemm_half(buf, s, rel):
            origin = (my + rel) % N_DEV
            blk = jnp.dot(buf[rel], w_ref[...],
                          preferred_element_type=jnp.float32)
            out_ref[pl.ds(origin * m_per + s * mh, mh), :] = blk
            return jnp.max(jnp.abs(blk))

        def mrc(buf, src_slot, dst_slot, send_sem, recv_sem, dev):
            return pltpu.make_async_remote_copy(
                src_ref=buf.at[src_slot],
                dst_ref=buf.at[dst_slot],
                send_sem=send_sem,
                recv_sem=recv_sem,
                device_id=(dev,),
                device_id_type=pl.DeviceIdType.MESH,
            )

        ra = [mrc(commA_ref, 0, 15, ra_ss.at[0], ra_rs.at[0], right)]
        rb = [mrc(commB_ref, 0, 15, rb_ss.at[0], rb_rs.at[0], right)]
        la = [mrc(commA_ref, 0, 1, la_ss.at[0], la_rs.at[0], left)]
        lb = [mrc(commB_ref, 0, 1, lb_ss.at[0], lb_rs.at[0], left)]
        for r in (ra[0], rb[0], la[0], lb[0]):
            r.start()

        blk0 = jnp.dot(x_ref[...], w_ref[...],
                       preferred_element_type=jnp.float32)
        out_ref[pl.ds(my * m_per, m_per), :] = blk0
        amax = jnp.max(jnp.abs(blk0))

        for h in range(8):
            ra[h].wait_recv()
            if h + 1 < RA:
                nxt = mrc(commA_ref, 15 - h, 14 - h,
                          ra_ss.at[h + 1], ra_rs.at[h + 1], right)
                nxt.start()
                ra.append(nxt)
            if h < RB:
                rb[h].wait_recv()
                if h + 1 < RB:
                    nxt = mrc(commB_ref, 15 - h, 14 - h,
                              rb_ss.at[h + 1], rb_rs.at[h + 1], right)
                    nxt.start()
                    rb.append(nxt)
            if h < LA:
                la[h].wait_recv()
                if h + 1 < LA:
                    nxt = mrc(commA_ref, 1 + h, 2 + h,
                              la_ss.at[h + 1], la_rs.at[h + 1], left)
                    nxt.start()
                    la.append(nxt)
            lb[h].wait_recv()
            if h + 1 < LB:
                nxt = mrc(commB_ref, 1 + h, 2 + h,
                          lb_ss.at[h + 1], lb_rs.at[h + 1], left)
                nxt.start()
                lb.append(nxt)
            amax = jnp.maximum(amax, gemm_half(commA_ref, 0, 15 - h))
            amax = jnp.maximum(amax, gemm_half(commB_ref, 1, 15 - h))
            if h < 7:
                amax = jnp.maximum(amax, gemm_half(commA_ref, 0, 1 + h))
                amax = jnp.maximum(amax, gemm_half(commB_ref, 1, 1 + h))

        for r in ra + rb + la + lb:
            r.wait_send()

        abuf_ref[...] = jnp.full((8, 128), amax, jnp.float32)
        gbuf_ref[0] = jnp.full((8, 128), amax, jnp.float32)

        amax_rdmas = []
        for dist in range(1, N_DEV):
            r = pltpu.make_async_remote_copy(
                src_ref=abuf_ref,
                dst_ref=gbuf_ref.at[N_DEV - dist],
                send_sem=a_send_sems.at[dist - 1],
                recv_sem=a_recv_sems.at[dist - 1],
                device_id=((my + dist) % N_DEV,),
                device_id_type=pl.DeviceIdType.MESH,
            )
            r.start()
            amax_rdmas.append(r)
        for r in amax_rdmas:
            r.wait()

        gmax = jnp.max(gbuf_ref[...])
        scale = gmax / 448.0
        q = (out_ref[...] / scale).astype(jnp.float8_e4m3fn)
        out_ref[...] = q.astype(jnp.float32) * scale

    return pl.pallas_call(
        body,
        out_shape=jax.ShapeDtypeStruct((m_tot, n_per), jnp.float32),
        in_specs=[
            pl.BlockSpec(memory_space=pltpu.VMEM),
            pl.BlockSpec(memory_space=pltpu.VMEM),
        ],
        out_specs=pl.BlockSpec(memory_space=pltpu.VMEM),
        scratch_shapes=[
            pltpu.VMEM((N_DEV, mh, k), jnp.bfloat16),
            pltpu.VMEM((N_DEV, mh, k), jnp.bfloat16),
            pltpu.VMEM((N_DEV, 8, 128), jnp.float32),
            pltpu.VMEM((8, 128), jnp.float32),
            pltpu.SemaphoreType.DMA((2,)),
            pltpu.SemaphoreType.DMA((RA,)),
            pltpu.SemaphoreType.DMA((RA,)),
            pltpu.SemaphoreType.DMA((RB,)),
            pltpu.SemaphoreType.DMA((RB,)),
            pltpu.SemaphoreType.DMA((LA,)),
            pltpu.SemaphoreType.DMA((LA,)),
            pltpu.SemaphoreType.DMA((LB,)),
            pltpu.SemaphoreType.DMA((LB,)),
            pltpu.SemaphoreType.DMA((N_DEV - 1,)),
            pltpu.SemaphoreType.DMA((N_DEV - 1,)),
        ],
        compiler_params=pltpu.CompilerParams(
            collective_id=0,
            vmem_limit_bytes=100 * 1024 * 1024,
        ),
    )(xb, wb)


# device time: 203165 ns/iter; 2.0658x vs baseline; 1.0334x over previous
import jax
import jax.numpy as jnp
from jax import lax
from jax.experimental import pallas as pl
from jax.experimental.pallas import tpu as pltpu

N_DEV = 16
RA, LA = 8, 7
RB, LB = 7, 8


def kernel(x, w_mat):
    m_per, k = x.shape
    _, n_per = w_mat.shape
    m_tot = N_DEV * m_per
    mh = m_per // 2

    def body(x_ref, w_ref, out_ref, commA_ref, commB_ref, wb_ref,
             gbuf_ref, abuf_ref,
             ra_ss, ra_rs, rb_ss, rb_rs, la_ss, la_rs, lb_ss, lb_rs,
             a_send_sems, a_recv_sems):
        my = lax.axis_index("i")
        left = (my - 1) % N_DEV
        right = (my + 1) % N_DEV

        barrier_sem = pltpu.get_barrier_semaphore()
        for nbr in (left, right):
            pl.semaphore_signal(
                barrier_sem, inc=1,
                device_id=(nbr,), device_id_type=pl.DeviceIdType.MESH,
            )
        pl.semaphore_wait(barrier_sem, 2)

        commA_ref[0] = x_ref[pl.ds(0, mh), :].astype(jnp.bfloat16)
        commB_ref[0] = x_ref[pl.ds(mh, mh), :].astype(jnp.bfloat16)

        def gemm_half(buf, s, rel):
            origin = (my + rel) % N_DEV
            blk = jnp.dot(buf[rel], wb_ref[...],
                          preferred_element_type=jnp.float32)
            out_ref[pl.ds(origin * m_per + s * mh, mh), :] = blk
            return jnp.max(jnp.abs(blk))

        def mrc(buf, src_slot, dst_slot, send_sem, recv_sem, dev):
            return pltpu.make_async_remote_copy(
                src_ref=buf.at[src_slot],
                dst_ref=buf.at[dst_slot],
                send_sem=send_sem,
                recv_sem=recv_sem,
                device_id=(dev,),
                device_id_type=pl.DeviceIdType.MESH,
            )

        ra = [mrc(commA_ref, 0, 15, ra_ss.at[0], ra_rs.at[0], right)]
        rb = [mrc(commB_ref, 0, 15, rb_ss.at[0], rb_rs.at[0], right)]
        la = [mrc(commA_ref, 0, 1, la_ss.at[0], la_rs.at[0], left)]
        lb = [mrc(commB_ref, 0, 1, lb_ss.at[0], lb_rs.at[0], left)]
        for r in (ra[0], rb[0], la[0], lb[0]):
            r.start()

        wb_ref[...] = w_ref[...].astype(jnp.bfloat16)
        amax = jnp.maximum(gemm_half(commA_ref, 0, 0),
                           gemm_half(commB_ref, 1, 0))

        for h in range(8):
            ra[h].wait_recv()
            if h + 1 < RA:
                nxt = mrc(commA_ref, 15 - h, 14 - h,
                          ra_ss.at[h + 1], ra_rs.at[h + 1], right)
                nxt.start()
                ra.append(nxt)
            if h < RB:
                rb[h].wait_recv()
                if h + 1 < RB:
                    nxt = mrc(commB_ref, 15 - h, 14 - h,
                              rb_ss.at[h + 1], rb_rs.at[h + 1], right)
                    nxt.start()
                    rb.append(nxt)
            if h < LA:
                la[h].wait_recv()
                if h + 1 < LA:
                    nxt = mrc(commA_ref, 1 + h, 2 + h,
                              la_ss.at[h + 1], la_rs.at[h + 1], left)
                    nxt.start()
                    la.append(nxt)
            lb[h].wait_recv()
            if h + 1 < LB:
                nxt = mrc(commB_ref, 1 + h, 2 + h,
                          lb_ss.at[h + 1], lb_rs.at[h + 1], left)
                nxt.start()
                lb.append(nxt)
            amax = jnp.maximum(amax, gemm_half(commA_ref, 0, 15 - h))
            amax = jnp.maximum(amax, gemm_half(commB_ref, 1, 15 - h))
            if h < 7:
                amax = jnp.maximum(amax, gemm_half(commA_ref, 0, 1 + h))
                amax = jnp.maximum(amax, gemm_half(commB_ref, 1, 1 + h))

        for r in ra + rb + la + lb:
            r.wait_send()

        abuf_ref[...] = jnp.full((8, 128), amax, jnp.float32)
        gbuf_ref[0] = jnp.full((8, 128), amax, jnp.float32)

        amax_rdmas = []
        for dist in range(1, N_DEV):
            r = pltpu.make_async_remote_copy(
                src_ref=abuf_ref,
                dst_ref=gbuf_ref.at[N_DEV - dist],
                send_sem=a_send_sems.at[dist - 1],
                recv_sem=a_recv_sems.at[dist - 1],
                device_id=((my + dist) % N_DEV,),
                device_id_type=pl.DeviceIdType.MESH,
            )
            r.start()
            amax_rdmas.append(r)
        for r in amax_rdmas:
            r.wait()

        gmax = jnp.max(gbuf_ref[...])
        scale = gmax / 448.0
        q = (out_ref[...] / scale).astype(jnp.float8_e4m3fn)
        out_ref[...] = q.astype(jnp.float32) * scale

    return pl.pallas_call(
        body,
        out_shape=jax.ShapeDtypeStruct((m_tot, n_per), jnp.float32),
        in_specs=[
            pl.BlockSpec(memory_space=pltpu.VMEM),
            pl.BlockSpec(memory_space=pltpu.VMEM),
        ],
        out_specs=pl.BlockSpec(memory_space=pltpu.VMEM),
        scratch_shapes=[
            pltpu.VMEM((N_DEV, mh, k), jnp.bfloat16),
            pltpu.VMEM((N_DEV, mh, k), jnp.bfloat16),
            pltpu.VMEM((k, n_per), jnp.bfloat16),
            pltpu.VMEM((N_DEV, 8, 128), jnp.float32),
            pltpu.VMEM((8, 128), jnp.float32),
            pltpu.SemaphoreType.DMA((RA,)),
            pltpu.SemaphoreType.DMA((RA,)),
            pltpu.SemaphoreType.DMA((RB,)),
            pltpu.SemaphoreType.DMA((RB,)),
            pltpu.SemaphoreType.DMA((LA,)),
            pltpu.SemaphoreType.DMA((LA,)),
            pltpu.SemaphoreType.DMA((LB,)),
            pltpu.SemaphoreType.DMA((LB,)),
            pltpu.SemaphoreType.DMA((N_DEV - 1,)),
            pltpu.SemaphoreType.DMA((N_DEV - 1,)),
        ],
        compiler_params=pltpu.CompilerParams(
            collective_id=0,
            vmem_limit_bytes=100 * 1024 * 1024,
        ),
    )(x, w_mat)


# device time: 198888 ns/iter; 2.1102x vs baseline; 1.0215x over previous
import jax
import jax.numpy as jnp
from jax import lax
from jax.experimental import pallas as pl
from jax.experimental.pallas import tpu as pltpu

N_DEV = 16
RA, LA = 8, 7
RB, LB = 7, 8


def kernel(x, w_mat):
    m_per, k = x.shape
    _, n_per = w_mat.shape
    m_tot = N_DEV * m_per
    mh = m_per // 2

    def body(x_ref, w_ref, out_ref, commA_ref, commB_ref, wb_ref,
             gbuf_ref, abuf_ref,
             ra_ss, ra_rs, rb_ss, rb_rs, la_ss, la_rs, lb_ss, lb_rs,
             a_send_sems, a_recv_sems):
        my = lax.axis_index("i")
        left = (my - 1) % N_DEV
        right = (my + 1) % N_DEV

        barrier_sem = pltpu.get_barrier_semaphore()
        for nbr in (left, right):
            pl.semaphore_signal(
                barrier_sem, inc=1,
                device_id=(nbr,), device_id_type=pl.DeviceIdType.MESH,
            )
        pl.semaphore_wait(barrier_sem, 2)

        commA_ref[0] = x_ref[pl.ds(0, mh), :].astype(jnp.bfloat16)
        commB_ref[0] = x_ref[pl.ds(mh, mh), :].astype(jnp.bfloat16)

        def gemm_half(buf, s, rel):
            origin = (my + rel) % N_DEV
            blk = jnp.dot(buf[rel], wb_ref[...],
                          preferred_element_type=jnp.float32)
            out_ref[pl.ds(origin * m_per + s * mh, mh), :] = blk
            return jnp.max(jnp.abs(blk))

        def mrc(buf, src_slot, dst_slot, send_sem, recv_sem, dev):
            return pltpu.make_async_remote_copy(
                src_ref=buf.at[src_slot],
                dst_ref=buf.at[dst_slot],
                send_sem=send_sem,
                recv_sem=recv_sem,
                device_id=(dev,),
                device_id_type=pl.DeviceIdType.MESH,
            )

        ra = [mrc(commA_ref, 0, 15, ra_ss.at[0], ra_rs.at[0], right)]
        rb = [mrc(commB_ref, 0, 15, rb_ss.at[0], rb_rs.at[0], right)]
        la = [mrc(commA_ref, 0, 1, la_ss.at[0], la_rs.at[0], left)]
        lb = [mrc(commB_ref, 0, 1, lb_ss.at[0], lb_rs.at[0], left)]
        for r in (ra[0], lb[0], rb[0], la[0]):
            r.start()

        wb_ref[...] = w_ref[...].astype(jnp.bfloat16)
        amax = jnp.maximum(gemm_half(commA_ref, 0, 0),
                           gemm_half(commB_ref, 1, 0))

        for h in range(8):
            ra[h].wait_recv()
            if h + 1 < RA:
                nxt = mrc(commA_ref, 15 - h, 14 - h,
                          ra_ss.at[h + 1], ra_rs.at[h + 1], right)
                nxt.start()
                ra.append(nxt)
            lb[h].wait_recv()
            if h + 1 < LB:
                nxt = mrc(commB_ref, 1 + h, 2 + h,
                          lb_ss.at[h + 1], lb_rs.at[h + 1], left)
                nxt.start()
                lb.append(nxt)
            if h < RB:
                rb[h].wait_recv()
                if h + 1 < RB:
                    nxt = mrc(commB_ref, 15 - h, 14 - h,
                              rb_ss.at[h + 1], rb_rs.at[h + 1], right)
                    nxt.start()
                    rb.append(nxt)
            if h < LA:
                la[h].wait_recv()
                if h + 1 < LA:
                    nxt = mrc(commA_ref, 1 + h, 2 + h,
                              la_ss.at[h + 1], la_rs.at[h + 1], left)
                    nxt.start()
                    la.append(nxt)
            amax = jnp.maximum(amax, gemm_half(commA_ref, 0, 15 - h))
            amax = jnp.maximum(amax, gemm_half(commB_ref, 1, 15 - h))
            if h < 7:
                amax = jnp.maximum(amax, gemm_half(commA_ref, 0, 1 + h))
                amax = jnp.maximum(amax, gemm_half(commB_ref, 1, 1 + h))

        for r in ra + rb + la + lb:
            r.wait_send()

        abuf_ref[...] = jnp.full((8, 128), amax, jnp.float32)
        gbuf_ref[0] = jnp.full((8, 128), amax, jnp.float32)

        amax_rdmas = []
        for dist in range(1, N_DEV):
            r = pltpu.make_async_remote_copy(
                src_ref=abuf_ref,
                dst_ref=gbuf_ref.at[N_DEV - dist],
                send_sem=a_send_sems.at[dist - 1],
                recv_sem=a_recv_sems.at[dist - 1],
                device_id=((my + dist) % N_DEV,),
                device_id_type=pl.DeviceIdType.MESH,
            )
            r.start()
            amax_rdmas.append(r)
        for r in amax_rdmas:
            r.wait()

        gmax = jnp.max(gbuf_ref[...])
        scale = gmax / 448.0
        inv_scale = 448.0 / gmax
        q = (out_ref[...] * inv_scale).astype(jnp.float8_e4m3fn)
        out_ref[...] = q.astype(jnp.float32) * scale

    return pl.pallas_call(
        body,
        out_shape=jax.ShapeDtypeStruct((m_tot, n_per), jnp.float32),
        in_specs=[
            pl.BlockSpec(memory_space=pltpu.VMEM),
            pl.BlockSpec(memory_space=pltpu.VMEM),
        ],
        out_specs=pl.BlockSpec(memory_space=pltpu.VMEM),
        scratch_shapes=[
            pltpu.VMEM((N_DEV, mh, k), jnp.bfloat16),
            pltpu.VMEM((N_DEV, mh, k), jnp.bfloat16),
            pltpu.VMEM((k, n_per), jnp.bfloat16),
            pltpu.VMEM((N_DEV, 8, 128), jnp.float32),
            pltpu.VMEM((8, 128), jnp.float32),
            pltpu.SemaphoreType.DMA((RA,)),
            pltpu.SemaphoreType.DMA((RA,)),
            pltpu.SemaphoreType.DMA((RB,)),
            pltpu.SemaphoreType.DMA((RB,)),
            pltpu.SemaphoreType.DMA((LA,)),
            pltpu.SemaphoreType.DMA((LA,)),
            pltpu.SemaphoreType.DMA((LB,)),
            pltpu.SemaphoreType.DMA((LB,)),
            pltpu.SemaphoreType.DMA((N_DEV - 1,)),
            pltpu.SemaphoreType.DMA((N_DEV - 1,)),
        ],
        compiler_params=pltpu.CompilerParams(
            collective_id=0,
            vmem_limit_bytes=100 * 1024 * 1024,
        ),
    )(x, w_mat)
